# baseline (speedup 1.0000x reference)
import jax
import jax.numpy as jnp
from jax import lax
from jax.experimental import pallas as pl
from jax.experimental.pallas import tpu as pltpu
from jax.experimental.pallas import tpu_sc as plsc

N = 320000
H = 128
B = 512

NC = 2
NS = 16
NW = NC * NS
ROWS_PER_W = N // NW
D = 80
NDMA = ROWS_PER_W // D
NPAIR = (NDMA - 1) // 2
RUNROLL = 5


def _sc_body(emb_hbm, ids_hbm, acc_out, cnt_out,
             ids_v, buf0, buf1, acc, cnt, dsem0, dsem1):
    cid = lax.axis_index("c")
    sid = lax.axis_index("s")
    wid = cid * NS + sid
    base = wid * ROWS_PER_W

    zeros16 = jnp.zeros((16,), jnp.float32)
    ones16 = jnp.ones((16,), jnp.float32)
    iota16 = lax.iota(jnp.int32, 16)

    pltpu.sync_copy(ids_hbm.at[wid], ids_v)

    def _zacc(r, c):
        for g in range(8):
            acc[pl.ds(r * H + g * 16, 16)] = zeros16
        cnt[pl.ds(r * 16, 16)] = zeros16
        return c
    lax.fori_loop(0, B, _zacc, 0)

    def _src(j):
        return emb_hbm.at[pl.ds(base + j * D, D)]

    def _consume(buf, j):
        def _rows(r16, c):
            idvec = ids_v[j, pl.ds(r16 * 16, 16)]
            ovec = idvec * H
            cvec = idvec * 16
            for u in range(16):
                r = r16 * 16 + u
                o = ovec[u]
                for g in range(8):
                    x = buf[r, pl.ds(g * 16, 16)]
                    plsc.addupdate_scatter(acc, [iota16 + (o + g * 16)], x)
                plsc.addupdate_scatter(cnt, [iota16 + cvec[u]], ones16)
            return c
        lax.fori_loop(0, D // 16, _rows, 0)

    pltpu.async_copy(_src(0), buf0, dsem0)

    def _pair(t, c):
        for b in (0, 1):
            buf, dsem = (buf0, dsem0) if b == 0 else (buf1, dsem1)
            obuf, odsem = (buf1, dsem1) if b == 0 else (buf0, dsem0)
            j = t * 2 + b
            pltpu.make_async_copy(_src(j), buf, dsem).wait()

            @pl.when(j + 1 < NDMA)
            def _():
                pltpu.async_copy(_src(j + 1), obuf, odsem)

            _consume(buf, j)
        return c
    lax.fori_loop(0, NPAIR, _pair, 0)

    pltpu.make_async_copy(_src(NDMA - 1), buf0, dsem0).wait()
    _consume(buf0, NDMA - 1)

    pltpu.sync_copy(acc, acc_out.at[wid])
    pltpu.sync_copy(cnt, cnt_out.at[wid])


def _make_sc_segsum(interpret=False):
    mesh = plsc.VectorSubcoreMesh(core_axis_name="c", subcore_axis_name="s")
    return pl.kernel(
        _sc_body,
        out_type=(
            jax.ShapeDtypeStruct((NW, B * H), jnp.float32),
            jax.ShapeDtypeStruct((NW, B * 16), jnp.float32),
        ),
        mesh=mesh,
        scratch_types=[
            pltpu.VMEM((NDMA, D), jnp.int32),
            pltpu.VMEM((D, H), jnp.float32),
            pltpu.VMEM((D, H), jnp.float32),
            pltpu.VMEM((B * H,), jnp.float32),
            pltpu.VMEM((B * 16,), jnp.float32),
            pltpu.SemaphoreType.DMA,
            pltpu.SemaphoreType.DMA,
        ],
        compiler_params=pltpu.CompilerParams(use_tc_tiling_on_sc=False,
                                             needs_layout_passes=False),
        interpret=interpret,
    )


def _tc_body(acc_ref, cnt_ref, w1_ref, b1_ref, w2_ref, b2_ref, o_ref):
    sums = jnp.sum(acc_ref[...], axis=0)
    cnts = jnp.sum(cnt_ref[...], axis=0)
    cnt = cnts[:, 0:1]
    mean = sums / jnp.maximum(cnt, 1.0)
    h = jnp.dot(mean, w1_ref[...], preferred_element_type=jnp.float32)
    h = jnp.maximum(h + b1_ref[...], 0.0)
    z = jnp.dot(h, w2_ref[...], preferred_element_type=jnp.float32)
    z = z + b2_ref[...]
    o_ref[...] = 1.0 / (1.0 + jnp.exp(-z))


def _tc_mlp(acc, cnt, W1, b1, W2, b2):
    return pl.pallas_call(
        _tc_body,
        out_shape=jax.ShapeDtypeStruct((B, 1), jnp.float32),
    )(acc, cnt, W1, b1, W2, b2)


@jax.jit
def kernel(v_embedding, segment_ids, W1, b1, W2, b2):
    ids = segment_ids.astype(jnp.int32).reshape(NW, NDMA, D)
    sc_fn = _make_sc_segsum()
    acc, cnt = sc_fn(v_embedding, ids)
    out = _tc_mlp(acc.reshape(NW, B, H), cnt.reshape(NW, B, 16),
                  W1, b1.reshape(1, H), W2, b2.reshape(1, 1))
    return out[:, 0]


# speedup vs baseline: 1.4708x; 1.4708x over previous
import jax
import jax.numpy as jnp
from jax import lax
from jax.experimental import pallas as pl
from jax.experimental.pallas import tpu as pltpu
from jax.experimental.pallas import tpu_sc as plsc

N = 320000
H = 128
HH = H // 2
B = 512

NC = 2
NS = 16
ROWS_PER_S = N // NS
D = 125
NDMA = ROWS_PER_S // D
NPAIR = NDMA // 2


def _sc_body(emb_hbm, ids_hbm, acc_out, cnt_out,
             ids_v, buf0, buf1, ones_v, zb, zbc, acc, cnt,
             dsem0, dsem1):
    cid = lax.axis_index("c")
    sid = lax.axis_index("s")
    wid = cid * NS + sid
    base = sid * ROWS_PER_S
    col0 = cid * HH
    duty = cid == 0

    zeros16 = jnp.zeros((16,), jnp.float32)
    ones16 = jnp.ones((16,), jnp.float32)

    pltpu.sync_copy(ids_hbm.at[sid], ids_v)

    def _zrow(r, c):
        for g in range(4):
            zb[r, pl.ds(g * 16, 16)] = zeros16
        zbc[r, :] = zeros16
        return c
    lax.fori_loop(0, 128, _zrow, 0)

    def _orow(r, c):
        ones_v[r, :] = ones16
        return c
    lax.fori_loop(0, D, _orow, 0)

    for qq in range(4):
        pltpu.sync_copy(zb, acc.at[pl.ds(qq * 128, 128)])
        pltpu.sync_copy(zbc, cnt.at[pl.ds(qq * 128, 128)])

    def _src(j):
        return emb_hbm.at[pl.ds(base + j * D, D), pl.ds(col0, HH)]

    pltpu.async_copy(_src(0), buf0, dsem0)
    pltpu.async_copy(_src(1), buf1, dsem1)

    def _pair(t, c):
        for b in (0, 1):
            buf, dsem = (buf0, dsem0) if b == 0 else (buf1, dsem1)
            j = t * 2 + b
            pltpu.make_async_copy(_src(j), buf, dsem).wait()
            idx = ids_v.at[j]
            pltpu.sync_copy(buf, acc.at[idx], add=True)

            @pl.when(duty)
            def _():
                pltpu.sync_copy(ones_v, cnt.at[idx], add=True)

            @pl.when(j + 2 < NDMA)
            def _():
                pltpu.async_copy(_src(j + 2), buf, dsem)
        return c
    lax.fori_loop(0, NPAIR, _pair, 0)

    pltpu.sync_copy(acc, acc_out.at[wid])

    @pl.when(duty)
    def _():
        pltpu.sync_copy(cnt, cnt_out.at[sid])


def _make_sc_segsum(interpret=False):
    mesh = plsc.VectorSubcoreMesh(core_axis_name="c", subcore_axis_name="s")
    return pl.kernel(
        _sc_body,
        out_type=(
            jax.ShapeDtypeStruct((NC * NS, B, HH), jnp.float32),
            jax.ShapeDtypeStruct((NS, B, 16), jnp.float32),
        ),
        mesh=mesh,
        scratch_types=[
            pltpu.VMEM((NDMA, D), jnp.int32),
            pltpu.VMEM((D, HH), jnp.float32),
            pltpu.VMEM((D, HH), jnp.float32),
            pltpu.VMEM((D, 16), jnp.float32),
            pltpu.VMEM((128, HH), jnp.float32),
            pltpu.VMEM((128, 16), jnp.float32),
            pltpu.VMEM_SHARED((B, HH), jnp.float32),
            pltpu.VMEM_SHARED((B, 16), jnp.float32),
            pltpu.SemaphoreType.DMA,
            pltpu.SemaphoreType.DMA,
        ],
        compiler_params=pltpu.CompilerParams(use_tc_tiling_on_sc=False),
        interpret=interpret,
    )


def _tc_body(acc_ref, cnt_ref, w1_ref, b1_ref, w2_ref, b2_ref, o_ref):
    left = jnp.sum(acc_ref[0], axis=0)
    right = jnp.sum(acc_ref[1], axis=0)
    sums = jnp.concatenate([left, right], axis=1)
    cnts = jnp.sum(cnt_ref[...], axis=0)
    cnt = cnts[:, 0:1]
    mean = sums / jnp.maximum(cnt, 1.0)
    h = jnp.dot(mean, w1_ref[...], preferred_element_type=jnp.float32)
    h = jnp.maximum(h + b1_ref[...], 0.0)
    z = jnp.dot(h, w2_ref[...], preferred_element_type=jnp.float32)
    z = z + b2_ref[...]
    o_ref[...] = 1.0 / (1.0 + jnp.exp(-z))


def _tc_mlp(acc, cnt, W1, b1, W2, b2):
    return pl.pallas_call(
        _tc_body,
        out_shape=jax.ShapeDtypeStruct((B, 1), jnp.float32),
    )(acc, cnt, W1, b1, W2, b2)


@jax.jit
def kernel(v_embedding, segment_ids, W1, b1, W2, b2):
    ids = segment_ids.astype(jnp.int32).reshape(NS, NDMA, D)
    sc_fn = _make_sc_segsum()
    acc, cnt = sc_fn(v_embedding, ids)
    out = _tc_mlp(acc.reshape(NC, NS, B, HH), cnt,
                  W1, b1.reshape(1, H), W2, b2.reshape(1, 1))
    return out[:, 0]
